# baseline (speedup 1.0000x reference)
import functools

import jax
import jax.numpy as jnp
from jax import lax
from jax.experimental import pallas as pl
from jax.experimental.pallas import tpu as pltpu
from jax.experimental.pallas import tpu_sc as plsc

_VOCAB = 32768
_B, _S = 4, 512
_N = _B * _S
_K = 1792
_NC, _NS, _L = 2, 16, 16
_NW = _NC * _NS
_RPW = (_N - _K) // _NW
_ROWS_BLK = 32


def _tc_body(ids_ref, vals_ref, alias_ref, out_ref):
    del alias_ref
    ids = ids_ref[...]
    vals = vals_ref[...]
    iota = jax.lax.broadcasted_iota(jnp.int32, (_ROWS_BLK, _VOCAB), 1)
    out_ref[...] = jnp.where(iota == ids, vals, jnp.float32(0.0))


def _sc_body(cols_hbm, svals_hbm, out_hbm, zbuf0, zbuf1, cols_v, vals_v,
             sem0, sem1):
    wid = lax.axis_index("s") * _NC + lax.axis_index("c")

    pltpu.sync_copy(cols_hbm.at[pl.ds(wid * _L, _L)], cols_v)
    pltpu.sync_copy(svals_hbm.at[pl.ds(wid * _L, _L)], vals_v)

    zbufs = (zbuf0, zbuf1)
    sems = (sem0, sem1)

    def _z(i, carry):
        b = i * (_L * 8)
        for u in range(8):
            zbuf0[pl.ds(b + u * _L, _L)] = jnp.zeros((_L,), jnp.float32)
            zbuf1[pl.ds(b + u * _L, _L)] = jnp.zeros((_L,), jnp.float32)
        return carry
    lax.fori_loop(0, _VOCAB // (_L * 8), _z, 0)

    cols = cols_v[...]
    vals = vals_v[...]
    lane = lax.iota(jnp.int32, _L)
    zeros16 = jnp.zeros((_L,), jnp.float32)

    base_row = _K + wid * _RPW
    copies = [None, None]
    for r in range(_RPW):
        b = r % 2
        if copies[b] is not None:
            copies[b].wait()
            plsc.store_scatter(zbufs[b], [cols], zeros16,
                               mask=lane == (r - 2))
        plsc.store_scatter(zbufs[b], [cols], vals, mask=lane == r)
        dst = out_hbm.at[pl.ds((base_row + r) * _VOCAB, _VOCAB)]
        copies[b] = pltpu.async_copy(zbufs[b], dst, sems[b])
    for cp in copies:
        if cp is not None:
            cp.wait()


_sc_fill = functools.partial(
    pl.kernel,
    out_type=jax.ShapeDtypeStruct((_N * _VOCAB,), jnp.float32),
    mesh=plsc.VectorSubcoreMesh(core_axis_name="c", subcore_axis_name="s"),
    compiler_params=pltpu.CompilerParams(needs_layout_passes=False),
    scratch_types=[
        pltpu.VMEM((_VOCAB,), jnp.float32),
        pltpu.VMEM((_VOCAB,), jnp.float32),
        pltpu.VMEM((_L,), jnp.int32),
        pltpu.VMEM((_L,), jnp.float32),
        pltpu.SemaphoreType.DMA,
        pltpu.SemaphoreType.DMA,
    ],
)(_sc_body)


def kernel(input_ids, attention_mask, boost):
    B, S = input_ids.shape
    ids32 = input_ids.astype(jnp.int32)
    ids = jnp.clip(ids32, 0, _VOCAB - 1).reshape(_N)
    valid = (attention_mask == 1) & (ids32 >= 0) & (ids32 < _VOCAB)
    vals = jnp.where(valid.reshape(_N), boost.astype(jnp.float32),
                     jnp.float32(0.0))

    cols = ids[_K:].reshape(_NW, _RPW)
    svals = vals[_K:].reshape(_NW, _RPW)
    pad = ((0, 0), (0, _L - _RPW))
    cols = jnp.pad(cols, pad).reshape(_NW * _L)
    svals = jnp.pad(svals, pad).reshape(_NW * _L)
    sc_out = _sc_fill(cols, svals).reshape(_N, _VOCAB)

    out = pl.pallas_call(
        _tc_body,
        grid=(_K // _ROWS_BLK,),
        in_specs=[
            pl.BlockSpec((_ROWS_BLK, 1), lambda i: (i, 0)),
            pl.BlockSpec((_ROWS_BLK, 1), lambda i: (i, 0)),
            pl.BlockSpec((8, 128), lambda i: (0, 0)),
        ],
        out_specs=pl.BlockSpec((_ROWS_BLK, _VOCAB), lambda i: (i, 0)),
        out_shape=jax.ShapeDtypeStruct((_N, _VOCAB), jnp.float32),
        input_output_aliases={2: 0},
        compiler_params=pltpu.CompilerParams(
            dimension_semantics=("arbitrary",)),
    )(ids[:_K, None], vals[:_K, None], sc_out)
    return out.reshape(B, S, _VOCAB)


# speedup vs baseline: 3.6004x; 3.6004x over previous
import functools

import jax
import jax.numpy as jnp
from jax import lax
from jax.experimental import pallas as pl
from jax.experimental.pallas import tpu as pltpu
from jax.experimental.pallas import tpu_sc as plsc

_VOCAB = 32768
_B, _S = 4, 512
_N = _B * _S
_K = 1792
_NC, _NS, _L = 2, 16, 16
_NW = _NC * _NS
_RPW = (_N - _K) // _NW
_ROWS_BLK = 32


def _tc_body(ids_ref, vals_ref, alias_ref, out_ref):
    del alias_ref
    ids = ids_ref[...]
    vals = vals_ref[...]
    iota = jax.lax.broadcasted_iota(jnp.int32, (_ROWS_BLK, _VOCAB), 1)
    out_ref[...] = jnp.where(iota == ids, vals, jnp.float32(0.0))


def _sc_body(cols_hbm, svals_hbm, out_hbm, zbuf0, zbuf1, cols_v, vals_v,
             sem0, sem1):
    wid = lax.axis_index("s") * _NC + lax.axis_index("c")

    pltpu.sync_copy(cols_hbm.at[pl.ds(wid * _L, _L)], cols_v)
    pltpu.sync_copy(svals_hbm.at[pl.ds(wid * _L, _L)], vals_v)

    zbufs = (zbuf0, zbuf1)
    sems = (sem0, sem1)

    def _z(i, carry):
        b = i * (_L * 8)
        for u in range(8):
            zbuf0[pl.ds(b + u * _L, _L)] = jnp.zeros((_L,), jnp.float32)
            zbuf1[pl.ds(b + u * _L, _L)] = jnp.zeros((_L,), jnp.float32)
        return carry
    lax.fori_loop(0, _VOCAB // (_L * 8), _z, 0)

    cols = cols_v[...]
    vals = vals_v[...]
    lane = lax.iota(jnp.int32, _L)
    zeros16 = jnp.zeros((_L,), jnp.float32)

    base_row = _K + wid * _RPW
    copies = [None, None]
    for r in range(_RPW):
        b = r % 2
        if copies[b] is not None:
            copies[b].wait()
            plsc.store_scatter(zbufs[b], [cols], zeros16,
                               mask=lane == (r - 2))
        plsc.store_scatter(zbufs[b], [cols], vals, mask=lane == r)
        dst = out_hbm.at[base_row + r]
        copies[b] = pltpu.async_copy(zbufs[b], dst, sems[b])
    for cp in copies:
        if cp is not None:
            cp.wait()


_sc_fill = functools.partial(
    pl.kernel,
    out_type=jax.ShapeDtypeStruct((_N, _VOCAB), jnp.float32),
    mesh=plsc.VectorSubcoreMesh(core_axis_name="c", subcore_axis_name="s"),
    compiler_params=pltpu.CompilerParams(needs_layout_passes=False),
    scratch_types=[
        pltpu.VMEM((_VOCAB,), jnp.float32),
        pltpu.VMEM((_VOCAB,), jnp.float32),
        pltpu.VMEM((_L,), jnp.int32),
        pltpu.VMEM((_L,), jnp.float32),
        pltpu.SemaphoreType.DMA,
        pltpu.SemaphoreType.DMA,
    ],
)(_sc_body)


def kernel(input_ids, attention_mask, boost):
    B, S = input_ids.shape
    ids32 = input_ids.astype(jnp.int32)
    ids = jnp.clip(ids32, 0, _VOCAB - 1).reshape(_N)
    valid = (attention_mask == 1) & (ids32 >= 0) & (ids32 < _VOCAB)
    vals = jnp.where(valid.reshape(_N), boost.astype(jnp.float32),
                     jnp.float32(0.0))

    cols = ids[_K:].reshape(_NW, _RPW)
    svals = vals[_K:].reshape(_NW, _RPW)
    pad = ((0, 0), (0, _L - _RPW))
    cols = jnp.pad(cols, pad).reshape(_NW * _L)
    svals = jnp.pad(svals, pad).reshape(_NW * _L)
    sc_out = _sc_fill(cols, svals)

    out = pl.pallas_call(
        _tc_body,
        grid=(_K // _ROWS_BLK,),
        in_specs=[
            pl.BlockSpec((_ROWS_BLK, 1), lambda i: (i, 0)),
            pl.BlockSpec((_ROWS_BLK, 1), lambda i: (i, 0)),
            pl.BlockSpec((8, 128), lambda i: (0, 0)),
        ],
        out_specs=pl.BlockSpec((_ROWS_BLK, _VOCAB), lambda i: (i, 0)),
        out_shape=jax.ShapeDtypeStruct((_N, _VOCAB), jnp.float32),
        input_output_aliases={2: 0},
        compiler_params=pltpu.CompilerParams(
            dimension_semantics=("arbitrary",)),
    )(ids[:_K, None], vals[:_K, None], sc_out)
    return out.reshape(B, S, _VOCAB)


# speedup vs baseline: 3.6063x; 1.0016x over previous
import functools

import jax
import jax.numpy as jnp
from jax import lax
from jax.experimental import pallas as pl
from jax.experimental.pallas import tpu as pltpu
from jax.experimental.pallas import tpu_sc as plsc

_VOCAB = 32768
_B, _S = 4, 512
_N = _B * _S
_K = 1984
_NC, _NS, _L = 2, 16, 16
_NW = _NC * _NS
_RPW = (_N - _K) // _NW
_ROWS_BLK = 32


def _tc_body(ids_ref, vals_ref, alias_ref, out_ref):
    del alias_ref
    ids = ids_ref[...]
    vals = vals_ref[...]
    iota = jax.lax.broadcasted_iota(jnp.int32, (_ROWS_BLK, _VOCAB), 1)
    out_ref[...] = jnp.where(iota == ids, vals, jnp.float32(0.0))


def _sc_body(cols_hbm, svals_hbm, out_hbm, zbuf0, zbuf1, cols_v, vals_v,
             sem0, sem1):
    wid = lax.axis_index("s") * _NC + lax.axis_index("c")

    pltpu.sync_copy(cols_hbm.at[pl.ds(wid * _L, _L)], cols_v)
    pltpu.sync_copy(svals_hbm.at[pl.ds(wid * _L, _L)], vals_v)

    zbufs = (zbuf0, zbuf1)
    sems = (sem0, sem1)

    def _z(i, carry):
        b = i * (_L * 8)
        for u in range(8):
            zbuf0[pl.ds(b + u * _L, _L)] = jnp.zeros((_L,), jnp.float32)
            zbuf1[pl.ds(b + u * _L, _L)] = jnp.zeros((_L,), jnp.float32)
        return carry
    lax.fori_loop(0, _VOCAB // (_L * 8), _z, 0)

    cols = cols_v[...]
    vals = vals_v[...]
    lane = lax.iota(jnp.int32, _L)
    zeros16 = jnp.zeros((_L,), jnp.float32)

    base_row = _K + wid * _RPW
    copies = [None, None]
    for r in range(_RPW):
        b = r % 2
        if copies[b] is not None:
            copies[b].wait()
            plsc.store_scatter(zbufs[b], [cols], zeros16,
                               mask=lane == (r - 2))
        plsc.store_scatter(zbufs[b], [cols], vals, mask=lane == r)
        dst = out_hbm.at[base_row + r]
        copies[b] = pltpu.async_copy(zbufs[b], dst, sems[b])
    for cp in copies:
        if cp is not None:
            cp.wait()


_sc_fill = functools.partial(
    pl.kernel,
    out_type=jax.ShapeDtypeStruct((_N, _VOCAB), jnp.float32),
    mesh=plsc.VectorSubcoreMesh(core_axis_name="c", subcore_axis_name="s"),
    compiler_params=pltpu.CompilerParams(needs_layout_passes=False),
    scratch_types=[
        pltpu.VMEM((_VOCAB,), jnp.float32),
        pltpu.VMEM((_VOCAB,), jnp.float32),
        pltpu.VMEM((_L,), jnp.int32),
        pltpu.VMEM((_L,), jnp.float32),
        pltpu.SemaphoreType.DMA,
        pltpu.SemaphoreType.DMA,
    ],
)(_sc_body)


def kernel(input_ids, attention_mask, boost):
    B, S = input_ids.shape
    ids32 = input_ids.astype(jnp.int32)
    ids = jnp.clip(ids32, 0, _VOCAB - 1).reshape(_N)
    valid = (attention_mask == 1) & (ids32 >= 0) & (ids32 < _VOCAB)
    vals = jnp.where(valid.reshape(_N), boost.astype(jnp.float32),
                     jnp.float32(0.0))

    cols = ids[_K:].reshape(_NW, _RPW)
    svals = vals[_K:].reshape(_NW, _RPW)
    pad = ((0, 0), (0, _L - _RPW))
    cols = jnp.pad(cols, pad).reshape(_NW * _L)
    svals = jnp.pad(svals, pad).reshape(_NW * _L)
    sc_out = _sc_fill(cols, svals)

    out = pl.pallas_call(
        _tc_body,
        grid=(_K // _ROWS_BLK,),
        in_specs=[
            pl.BlockSpec((_ROWS_BLK, 1), lambda i: (i, 0)),
            pl.BlockSpec((_ROWS_BLK, 1), lambda i: (i, 0)),
            pl.BlockSpec((8, 128), lambda i: (0, 0)),
        ],
        out_specs=pl.BlockSpec((_ROWS_BLK, _VOCAB), lambda i: (i, 0)),
        out_shape=jax.ShapeDtypeStruct((_N, _VOCAB), jnp.float32),
        input_output_aliases={2: 0},
        compiler_params=pltpu.CompilerParams(
            dimension_semantics=("arbitrary",)),
    )(ids[:_K, None], vals[:_K, None], sc_out)
    return out.reshape(B, S, _VOCAB)


# speedup vs baseline: 3.6238x; 1.0049x over previous
import functools

import jax
import jax.numpy as jnp
from jax import lax
from jax.experimental import pallas as pl
from jax.experimental.pallas import tpu as pltpu
from jax.experimental.pallas import tpu_sc as plsc

_VOCAB = 32768
_B, _S = 4, 512
_N = _B * _S
_K = 1792
_NC, _NS, _L = 2, 16, 16
_NW = _NC * _NS
_RPW = (_N - _K) // _NW
_ROWS_BLK = 32


def _tc_body(ids_ref, vals_ref, alias_ref, out_ref):
    del alias_ref
    ids = ids_ref[...]
    vals = vals_ref[...]
    iota = jax.lax.broadcasted_iota(jnp.int32, (_ROWS_BLK, _VOCAB), 1)
    out_ref[...] = jnp.where(iota == ids, vals, jnp.float32(0.0))


def _sc_body(cols_hbm, svals_hbm, out_hbm, zbuf0, zbuf1, cols_v, vals_v,
             sem0, sem1):
    wid = lax.axis_index("s") * _NC + lax.axis_index("c")

    stage0 = pltpu.async_copy(cols_hbm.at[pl.ds(wid * _L, _L)], cols_v,
                              sem0)
    stage1 = pltpu.async_copy(svals_hbm.at[pl.ds(wid * _L, _L)], vals_v,
                              sem1)

    zbufs = (zbuf0, zbuf1)
    sems = (sem0, sem1)

    def _z(i, carry):
        b = i * (_L * 8)
        for u in range(8):
            zbuf0[pl.ds(b + u * _L, _L)] = jnp.zeros((_L,), jnp.float32)
            zbuf1[pl.ds(b + u * _L, _L)] = jnp.zeros((_L,), jnp.float32)
        return carry
    lax.fori_loop(0, _VOCAB // (_L * 8), _z, 0)
    stage0.wait()
    stage1.wait()

    cols = cols_v[...]
    vals = vals_v[...]
    lane = lax.iota(jnp.int32, _L)
    zeros16 = jnp.zeros((_L,), jnp.float32)

    base_row = _K + wid * _RPW
    copies = [None, None]
    for r in range(_RPW):
        b = r % 2
        if copies[b] is not None:
            copies[b].wait()
            plsc.store_scatter(zbufs[b], [cols], zeros16,
                               mask=lane == (r - 2))
        plsc.store_scatter(zbufs[b], [cols], vals, mask=lane == r)
        dst = out_hbm.at[base_row + r]
        copies[b] = pltpu.async_copy(zbufs[b], dst, sems[b])
    for cp in copies:
        if cp is not None:
            cp.wait()


_sc_fill = functools.partial(
    pl.kernel,
    out_type=jax.ShapeDtypeStruct((_N, _VOCAB), jnp.float32),
    mesh=plsc.VectorSubcoreMesh(core_axis_name="c", subcore_axis_name="s"),
    compiler_params=pltpu.CompilerParams(needs_layout_passes=False),
    scratch_types=[
        pltpu.VMEM((_VOCAB,), jnp.float32),
        pltpu.VMEM((_VOCAB,), jnp.float32),
        pltpu.VMEM((_L,), jnp.int32),
        pltpu.VMEM((_L,), jnp.float32),
        pltpu.SemaphoreType.DMA,
        pltpu.SemaphoreType.DMA,
    ],
)(_sc_body)


def kernel(input_ids, attention_mask, boost):
    B, S = input_ids.shape
    ids32 = input_ids.astype(jnp.int32)
    ids = jnp.clip(ids32, 0, _VOCAB - 1).reshape(_N)
    valid = (attention_mask == 1) & (ids32 >= 0) & (ids32 < _VOCAB)
    vals = jnp.where(valid.reshape(_N), boost.astype(jnp.float32),
                     jnp.float32(0.0))

    cols = ids[_K:].reshape(_NW, _RPW)
    svals = vals[_K:].reshape(_NW, _RPW)
    pad = ((0, 0), (0, _L - _RPW))
    cols = jnp.pad(cols, pad).reshape(_NW * _L)
    svals = jnp.pad(svals, pad).reshape(_NW * _L)
    sc_out = _sc_fill(cols, svals)

    out = pl.pallas_call(
        _tc_body,
        grid=(_K // _ROWS_BLK,),
        in_specs=[
            pl.BlockSpec((_ROWS_BLK, 1), lambda i: (i, 0)),
            pl.BlockSpec((_ROWS_BLK, 1), lambda i: (i, 0)),
            pl.BlockSpec((8, 128), lambda i: (0, 0)),
        ],
        out_specs=pl.BlockSpec((_ROWS_BLK, _VOCAB), lambda i: (i, 0)),
        out_shape=jax.ShapeDtypeStruct((_N, _VOCAB), jnp.float32),
        input_output_aliases={2: 0},
        compiler_params=pltpu.CompilerParams(
            dimension_semantics=("arbitrary",)),
    )(ids[:_K, None], vals[:_K, None], sc_out)
    return out.reshape(B, S, _VOCAB)


# speedup vs baseline: 3.6365x; 1.0035x over previous
import functools

import jax
import jax.numpy as jnp
from jax import lax
from jax.experimental import pallas as pl
from jax.experimental.pallas import tpu as pltpu
from jax.experimental.pallas import tpu_sc as plsc

_VOCAB = 32768
_B, _S = 4, 512
_N = _B * _S
_K = 1664
_NC, _NS, _L = 2, 16, 16
_NW = _NC * _NS
_RPW = (_N - _K) // _NW
_ROWS_BLK = 32


def _tc_body(ids_ref, vals_ref, alias_ref, out_ref):
    del alias_ref
    ids = ids_ref[...]
    vals = vals_ref[...]
    iota = jax.lax.broadcasted_iota(jnp.int32, (_ROWS_BLK, _VOCAB), 1)
    out_ref[...] = jnp.where(iota == ids, vals, jnp.float32(0.0))


def _sc_body(cols_hbm, svals_hbm, out_hbm, zbuf0, zbuf1, cols_v, vals_v,
             sem0, sem1):
    wid = lax.axis_index("s") * _NC + lax.axis_index("c")

    stage0 = pltpu.async_copy(cols_hbm.at[pl.ds(wid * _L, _L)], cols_v,
                              sem0)
    stage1 = pltpu.async_copy(svals_hbm.at[pl.ds(wid * _L, _L)], vals_v,
                              sem1)

    zbufs = (zbuf0, zbuf1)
    sems = (sem0, sem1)

    def _z(i, carry):
        b = i * (_L * 8)
        for u in range(8):
            zbuf0[pl.ds(b + u * _L, _L)] = jnp.zeros((_L,), jnp.float32)
            zbuf1[pl.ds(b + u * _L, _L)] = jnp.zeros((_L,), jnp.float32)
        return carry
    lax.fori_loop(0, _VOCAB // (_L * 8), _z, 0)
    stage0.wait()
    stage1.wait()

    cols = cols_v[...]
    vals = vals_v[...]
    lane = lax.iota(jnp.int32, _L)
    zeros16 = jnp.zeros((_L,), jnp.float32)

    base_row = _K + wid * _RPW
    copies = [None, None]
    for r in range(_RPW):
        b = r % 2
        if copies[b] is not None:
            copies[b].wait()
            plsc.store_scatter(zbufs[b], [cols], zeros16,
                               mask=lane == (r - 2))
        plsc.store_scatter(zbufs[b], [cols], vals, mask=lane == r)
        dst = out_hbm.at[base_row + r]
        copies[b] = pltpu.async_copy(zbufs[b], dst, sems[b])
    for cp in copies:
        if cp is not None:
            cp.wait()


_sc_fill = functools.partial(
    pl.kernel,
    out_type=jax.ShapeDtypeStruct((_N, _VOCAB), jnp.float32),
    mesh=plsc.VectorSubcoreMesh(core_axis_name="c", subcore_axis_name="s"),
    compiler_params=pltpu.CompilerParams(needs_layout_passes=False),
    scratch_types=[
        pltpu.VMEM((_VOCAB,), jnp.float32),
        pltpu.VMEM((_VOCAB,), jnp.float32),
        pltpu.VMEM((_L,), jnp.int32),
        pltpu.VMEM((_L,), jnp.float32),
        pltpu.SemaphoreType.DMA,
        pltpu.SemaphoreType.DMA,
    ],
)(_sc_body)


def kernel(input_ids, attention_mask, boost):
    B, S = input_ids.shape
    ids32 = input_ids.astype(jnp.int32)
    ids = jnp.clip(ids32, 0, _VOCAB - 1).reshape(_N)
    valid = (attention_mask == 1) & (ids32 >= 0) & (ids32 < _VOCAB)
    vals = jnp.where(valid.reshape(_N), boost.astype(jnp.float32),
                     jnp.float32(0.0))

    cols = ids[_K:].reshape(_NW, _RPW)
    svals = vals[_K:].reshape(_NW, _RPW)
    pad = ((0, 0), (0, _L - _RPW))
    cols = jnp.pad(cols, pad).reshape(_NW * _L)
    svals = jnp.pad(svals, pad).reshape(_NW * _L)
    sc_out = _sc_fill(cols, svals)

    out = pl.pallas_call(
        _tc_body,
        grid=(_K // _ROWS_BLK,),
        in_specs=[
            pl.BlockSpec((_ROWS_BLK, 1), lambda i: (i, 0)),
            pl.BlockSpec((_ROWS_BLK, 1), lambda i: (i, 0)),
            pl.BlockSpec((8, 128), lambda i: (0, 0)),
        ],
        out_specs=pl.BlockSpec((_ROWS_BLK, _VOCAB), lambda i: (i, 0)),
        out_shape=jax.ShapeDtypeStruct((_N, _VOCAB), jnp.float32),
        input_output_aliases={2: 0},
        compiler_params=pltpu.CompilerParams(
            dimension_semantics=("arbitrary",)),
    )(ids[:_K, None], vals[:_K, None], sc_out)
    return out.reshape(B, S, _VOCAB)
